# baseline (speedup 1.0000x reference)
import functools

import jax
import jax.numpy as jnp
from jax import lax
from jax.experimental import pallas as pl
from jax.experimental.pallas import tpu as pltpu
from jax.experimental.pallas import tpu_sc as plsc

VOCAB = 100000
HIDDEN = 768
EPS = 1e-3

NUM_SC_CORES = 2
NUM_SC_SUBCORES = 16
NUM_WORKERS = NUM_SC_CORES * NUM_SC_SUBCORES

CHUNK = 64

SEQ_SPLITS = (1024, 512, 512)


def _sc_gather(table, input_ids, seq_off, seq_chunk):
    batch, _ = input_ids.shape
    num_rows = batch * seq_chunk
    rows_per_worker = num_rows // NUM_WORKERS
    workers_per_batch = seq_chunk // rows_per_worker
    chunk = min(CHUNK, rows_per_worker // 2)
    n_chunks = rows_per_worker // chunk
    mesh = plsc.VectorSubcoreMesh(
        core_axis_name="c", subcore_axis_name="s",
        num_cores=NUM_SC_CORES, num_subcores=NUM_SC_SUBCORES)

    @functools.partial(
        pl.kernel,
        out_type=jax.ShapeDtypeStruct((num_rows, HIDDEN), jnp.float32),
        mesh=mesh,
        scratch_types=[
            pltpu.VMEM((rows_per_worker,), jnp.int32),
            pltpu.VMEM((chunk, HIDDEN), jnp.float32),
            pltpu.VMEM((chunk, HIDDEN), jnp.float32),
            pltpu.SemaphoreType.DMA,
            pltpu.SemaphoreType.DMA,
            pltpu.SemaphoreType.DMA,
            pltpu.SemaphoreType.DMA,
        ],
    )
    def gather_kernel(table_hbm, idx_hbm, out_hbm, idx_v, rows_a, rows_b,
                      gsem_a, gsem_b, wsem_a, wsem_b):
        wid = lax.axis_index("s") * NUM_SC_CORES + lax.axis_index("c")
        base = wid * rows_per_worker
        b = wid // workers_per_batch
        soff = seq_off + (wid % workers_per_batch) * rows_per_worker
        pltpu.sync_copy(idx_hbm.at[b, pl.ds(soff, rows_per_worker)], idx_v)

        bufs = (rows_a, rows_b)
        gsems = (gsem_a, gsem_b)
        wsems = (wsem_a, wsem_b)

        def start_gather(k):
            return pltpu.async_copy(
                table_hbm.at[idx_v.at[pl.ds(k * chunk, chunk)]],
                bufs[k % 2], gsems[k % 2])

        def start_write(k):
            return pltpu.async_copy(
                bufs[k % 2], out_hbm.at[pl.ds(base + k * chunk, chunk)],
                wsems[k % 2])

        g = {0: start_gather(0)}
        if n_chunks > 1:
            g[1] = start_gather(1)
        w = {}
        for k in range(n_chunks):
            g[k].wait()
            w[k] = start_write(k)
            if k + 2 < n_chunks:
                w[k].wait()
                g[k + 2] = start_gather(k + 2)
        for k in range(max(0, n_chunks - 2), n_chunks):
            w[k].wait()

    return gather_kernel(table, input_ids)


def _tc_add_pos_layernorm(gathered, pos, gamma, beta, seq_len, batch,
                          seq_off, seq_chunk, prev):
    bpb = 2
    pos_block_off = seq_off // seq_chunk

    def body(*refs):
        x_ref, p_ref, g_ref, b_ref = refs[:4]
        o_ref = refs[-1]
        x = x_ref[...].reshape(bpb, seq_chunk, HIDDEN) + p_ref[...][None]
        mean = jnp.mean(x, axis=-1, keepdims=True)
        xc = x - mean
        var = jnp.mean(xc * xc, axis=-1, keepdims=True)
        o_ref[...] = (xc * lax.rsqrt(var + EPS)) * g_ref[...] + b_ref[...]

    in_specs = [
        pl.BlockSpec((bpb * seq_chunk, HIDDEN), lambda b: (b, 0)),
        pl.BlockSpec((seq_chunk, HIDDEN), lambda b: (pos_block_off, 0)),
        pl.BlockSpec((1, HIDDEN), lambda b: (0, 0)),
        pl.BlockSpec((1, HIDDEN), lambda b: (0, 0)),
    ]
    args = [gathered, pos, gamma, beta]
    aliases = {}
    if prev is not None:
        in_specs.append(pl.BlockSpec(memory_space=pl.MemorySpace.ANY))
        args.append(prev)
        aliases = {4: 0}
    return pl.pallas_call(
        body,
        grid=(batch // bpb,),
        in_specs=in_specs,
        out_specs=pl.BlockSpec(
            (bpb, seq_chunk, HIDDEN),
            lambda b: (b, pos_block_off, 0)),
        out_shape=jax.ShapeDtypeStruct((batch, seq_len, HIDDEN), jnp.float32),
        input_output_aliases=aliases,
    )(*args)


def kernel(input_ids, token_type_ids, word_embeddings, position_embeddings,
           gamma, beta):
    batch, seq = input_ids.shape
    pos = position_embeddings[:seq]
    gamma2 = gamma.reshape(1, HIDDEN)
    beta2 = beta.reshape(1, HIDDEN)
    offs = [sum(SEQ_SPLITS[:i]) for i in range(len(SEQ_SPLITS))]
    gathered = [
        _sc_gather(word_embeddings, input_ids, off, sc)
        for off, sc in zip(offs, SEQ_SPLITS)
    ]
    out = None
    for (off, sc), g in zip(zip(offs, SEQ_SPLITS), gathered):
        out = _tc_add_pos_layernorm(g, pos, gamma2, beta2, seq, batch,
                                    off, sc, out)
    return out


# speedup vs baseline: 1.0414x; 1.0414x over previous
import functools

import jax
import jax.numpy as jnp
from jax import lax
from jax.experimental import pallas as pl
from jax.experimental.pallas import tpu as pltpu
from jax.experimental.pallas import tpu_sc as plsc

VOCAB = 100000
HIDDEN = 768
EPS = 1e-3

NUM_SC_CORES = 2
NUM_SC_SUBCORES = 16
NUM_WORKERS = NUM_SC_CORES * NUM_SC_SUBCORES

CHUNK = 64

SEQ_SPLITS = (1024, 1024)


def _sc_gather(table, input_ids, seq_off, seq_chunk):
    batch, _ = input_ids.shape
    num_rows = batch * seq_chunk
    rows_per_worker = num_rows // NUM_WORKERS
    workers_per_batch = seq_chunk // rows_per_worker
    chunk = min(CHUNK, rows_per_worker // 2)
    n_chunks = rows_per_worker // chunk
    mesh = plsc.VectorSubcoreMesh(
        core_axis_name="c", subcore_axis_name="s",
        num_cores=NUM_SC_CORES, num_subcores=NUM_SC_SUBCORES)

    @functools.partial(
        pl.kernel,
        out_type=jax.ShapeDtypeStruct((num_rows, HIDDEN), jnp.float32),
        mesh=mesh,
        scratch_types=[
            pltpu.VMEM((rows_per_worker,), jnp.int32),
            pltpu.VMEM((chunk, HIDDEN), jnp.float32),
            pltpu.VMEM((chunk, HIDDEN), jnp.float32),
            pltpu.SemaphoreType.DMA,
            pltpu.SemaphoreType.DMA,
            pltpu.SemaphoreType.DMA,
            pltpu.SemaphoreType.DMA,
        ],
    )
    def gather_kernel(table_hbm, idx_hbm, out_hbm, idx_v, rows_a, rows_b,
                      gsem_a, gsem_b, wsem_a, wsem_b):
        wid = lax.axis_index("s") * NUM_SC_CORES + lax.axis_index("c")
        base = wid * rows_per_worker
        b = wid // workers_per_batch
        soff = seq_off + (wid % workers_per_batch) * rows_per_worker
        pltpu.sync_copy(idx_hbm.at[b, pl.ds(soff, rows_per_worker)], idx_v)

        bufs = (rows_a, rows_b)
        gsems = (gsem_a, gsem_b)
        wsems = (wsem_a, wsem_b)

        def start_gather(k):
            return pltpu.async_copy(
                table_hbm.at[idx_v.at[pl.ds(k * chunk, chunk)]],
                bufs[k % 2], gsems[k % 2])

        def start_write(k):
            return pltpu.async_copy(
                bufs[k % 2], out_hbm.at[pl.ds(base + k * chunk, chunk)],
                wsems[k % 2])

        g = {0: start_gather(0)}
        if n_chunks > 1:
            g[1] = start_gather(1)
        w = {}
        for k in range(n_chunks):
            g[k].wait()
            w[k] = start_write(k)
            if k + 2 < n_chunks:
                w[k].wait()
                g[k + 2] = start_gather(k + 2)
        for k in range(max(0, n_chunks - 2), n_chunks):
            w[k].wait()

    return gather_kernel(table, input_ids)


def _tc_add_pos_layernorm(gathered, pos, gamma, beta, seq_len, batch,
                          seq_off, seq_chunk, prev):
    bpb = 2
    pos_block_off = seq_off // seq_chunk

    def body(*refs):
        x_ref, p_ref, g_ref, b_ref = refs[:4]
        o_ref = refs[-1]
        x = x_ref[...].reshape(bpb, seq_chunk, HIDDEN) + p_ref[...][None]
        mean = jnp.mean(x, axis=-1, keepdims=True)
        xc = x - mean
        var = jnp.mean(xc * xc, axis=-1, keepdims=True)
        o_ref[...] = (xc * lax.rsqrt(var + EPS)) * g_ref[...] + b_ref[...]

    in_specs = [
        pl.BlockSpec((bpb * seq_chunk, HIDDEN), lambda b: (b, 0)),
        pl.BlockSpec((seq_chunk, HIDDEN), lambda b: (pos_block_off, 0)),
        pl.BlockSpec((1, HIDDEN), lambda b: (0, 0)),
        pl.BlockSpec((1, HIDDEN), lambda b: (0, 0)),
    ]
    args = [gathered, pos, gamma, beta]
    aliases = {}
    if prev is not None:
        in_specs.append(pl.BlockSpec(memory_space=pl.MemorySpace.ANY))
        args.append(prev)
        aliases = {4: 0}
    return pl.pallas_call(
        body,
        grid=(batch // bpb,),
        in_specs=in_specs,
        out_specs=pl.BlockSpec(
            (bpb, seq_chunk, HIDDEN),
            lambda b: (b, pos_block_off, 0)),
        out_shape=jax.ShapeDtypeStruct((batch, seq_len, HIDDEN), jnp.float32),
        input_output_aliases=aliases,
    )(*args)


def kernel(input_ids, token_type_ids, word_embeddings, position_embeddings,
           gamma, beta):
    batch, seq = input_ids.shape
    pos = position_embeddings[:seq]
    gamma2 = gamma.reshape(1, HIDDEN)
    beta2 = beta.reshape(1, HIDDEN)
    offs = [sum(SEQ_SPLITS[:i]) for i in range(len(SEQ_SPLITS))]
    gathered = [
        _sc_gather(word_embeddings, input_ids, off, sc)
        for off, sc in zip(offs, SEQ_SPLITS)
    ]
    out = None
    for (off, sc), g in zip(zip(offs, SEQ_SPLITS), gathered):
        out = _tc_add_pos_layernorm(g, pos, gamma2, beta2, seq, batch,
                                    off, sc, out)
    return out
